# baseline (speedup 1.0000x reference)
import functools

import numpy as np

import jax
import jax.numpy as jnp
from jax import lax
from jax.experimental import pallas as pl
from jax.experimental.pallas import tpu as pltpu
from jax.experimental.pallas import tpu_sc as plsc

D = 64
R = 16
P = 512

_TW = 8192
_QW = _TW // 4
_MASK_HI = np.uint32(0xFFFF0000)
_RND = np.uint32(0x8000)

_NC = 2
_NS = 16
_NW = _NC * _NS
_GCHUNK = 128
_PHASE = 512


def _pack_pair(a_u32, b_u32):
  return (((a_u32 + _RND) & _MASK_HI)
          | ((b_u32 + _RND) >> np.uint32(16)))


def _transpose_body(x_ref, o_ref):
  x = x_ref[...]
  z = jnp.concatenate(
      [x[:, :_QW], x[:, _QW:2 * _QW], x[:, 2 * _QW:3 * _QW], x[:, 3 * _QW:]],
      axis=0)
  t = z.T
  u = lax.bitcast_convert_type(t, jnp.uint32)
  lo = _pack_pair(u[:, 0:D], u[:, D:2 * D])
  hi = _pack_pair(u[:, 2 * D:3 * D], u[:, 3 * D:])
  packed = jnp.concatenate([lo, hi], axis=1)
  o_ref[...] = lax.bitcast_convert_type(packed, jnp.float32)


def _make_pack(n_cols):
  n_blocks = (n_cols + _TW - 1) // _TW
  return pl.pallas_call(
      _transpose_body,
      grid=(n_blocks,),
      in_specs=[pl.BlockSpec((D, _TW), lambda j: (0, j))],
      out_specs=pl.BlockSpec((_QW, 2 * D), lambda j: (j, 0)),
      out_shape=jax.ShapeDtypeStruct((n_blocks * _QW, 2 * D), jnp.float32),
      compiler_params=pltpu.CompilerParams(
          dimension_semantics=("parallel",)),
  )


def _make_sc_gather(n_idx):
  b_per_w = n_idx // _NW
  mesh = plsc.VectorSubcoreMesh(core_axis_name="c", subcore_axis_name="s")

  @functools.partial(
      pl.kernel,
      mesh=mesh,
      out_type=jax.ShapeDtypeStruct((n_idx, 2 * D), jnp.float32),
      scratch_types=[
          pltpu.VMEM((b_per_w,), jnp.int32),
          pltpu.VMEM((_PHASE, 2 * D), jnp.float32),
          pltpu.SemaphoreType.DMA,
      ],
  )
  def gather_kernel(table_hbm, idx_hbm, out_hbm, idx_v, rows_v, sem):
    wid = lax.axis_index("s") * _NC + lax.axis_index("c")
    base = wid * b_per_w
    pltpu.sync_copy(idx_hbm.at[pl.ds(base, b_per_w)], idx_v)
    for ph in range(b_per_w // _PHASE):
      copies = []
      for j in range(_PHASE // _GCHUNK):
        isl = pl.ds(ph * _PHASE + j * _GCHUNK, _GCHUNK)
        rsl = pl.ds(j * _GCHUNK, _GCHUNK)
        copies.append(
            pltpu.async_copy(table_hbm.at[idx_v.at[isl]], rows_v.at[rsl],
                             sem))
      for c in copies:
        c.wait()
      pltpu.sync_copy(rows_v, out_hbm.at[pl.ds(base + ph * _PHASE, _PHASE)])

  return gather_kernel


def _unpack_side(g, idx):
  half = ((idx >> 12) & 1)[:, None] == 1
  slot = ((idx >> 11) & 1)[:, None] == 1
  gh = jnp.where(half, g[:, D:], g[:, :D])
  u = lax.bitcast_convert_type(gh, jnp.uint32)
  va = lax.bitcast_convert_type(u & _MASK_HI, jnp.float32)
  vb = lax.bitcast_convert_type(u << np.uint32(16), jnp.float32)
  return jnp.where(slot, vb, va)


def _score_body(lhs_ref, rhs_ref, lidx_ref, ridx_ref, relidx_ref, diag_ref,
                pos_ref, ln_ref, rn_ref):
  lhs = _unpack_side(lhs_ref[0], lidx_ref[0, 0])
  rhs = _unpack_side(rhs_ref[0], ridx_ref[0, 0])
  rel = relidx_ref[0, 0]
  onehot = (rel[:, None] == lax.broadcasted_iota(jnp.int32, (P, R), 1))
  diag = jnp.dot(onehot.astype(jnp.float32), diag_ref[...],
                 preferred_element_type=jnp.float32)
  rhs_t = rhs * diag
  pos_ref[0] = jnp.sum(lhs * rhs_t, axis=1)[None, :]
  lhs_bf = lhs.astype(jnp.bfloat16)
  rhs_bf = rhs_t.astype(jnp.bfloat16)
  dn = (((1,), (1,)), ((), ()))
  ln_ref[0] = lax.dot_general(rhs_bf, lhs_bf, dn,
                              preferred_element_type=jnp.float32)
  rn_ref[0] = lax.dot_general(lhs_bf, rhs_bf, dn,
                              preferred_element_type=jnp.float32)


def _make_tc_scores(C):
  return pl.pallas_call(
      _score_body,
      grid=(C,),
      in_specs=[
          pl.BlockSpec((1, P, 2 * D), lambda c: (c, 0, 0)),
          pl.BlockSpec((1, P, 2 * D), lambda c, _C=C: (_C + c, 0, 0)),
          pl.BlockSpec((1, 1, P), lambda c: (c, 0, 0)),
          pl.BlockSpec((1, 1, P), lambda c, _C=C: (_C + c, 0, 0)),
          pl.BlockSpec((1, 1, P), lambda c: (c, 0, 0)),
          pl.BlockSpec((R, D), lambda c: (0, 0)),
      ],
      out_specs=[
          pl.BlockSpec((1, 1, P), lambda c: (c, 0, 0)),
          pl.BlockSpec((1, P, P), lambda c: (c, 0, 0)),
          pl.BlockSpec((1, P, P), lambda c: (c, 0, 0)),
      ],
      out_shape=[
          jax.ShapeDtypeStruct((C, 1, P), jnp.float32),
          jax.ShapeDtypeStruct((C, P, P), jnp.float32),
          jax.ShapeDtypeStruct((C, P, P), jnp.float32),
      ],
      compiler_params=pltpu.CompilerParams(
          dimension_semantics=("parallel",)),
  )


@jax.jit
def kernel(emb, rel_diag, lhs_idx, rhs_idx, rel_idx):
  B = lhs_idx.shape[0]
  C = B // P
  V = emb.shape[0]
  packed = _make_pack(V)(emb.T)
  idx_all = jnp.concatenate([lhs_idx, rhs_idx]).astype(jnp.int32)
  gidx = ((idx_all >> 13) << 11) | (idx_all & 2047)
  gathered = _make_sc_gather(2 * B)(packed, gidx)
  g3 = gathered.reshape(2 * C, P, 2 * D)
  idx3 = idx_all.reshape(2 * C, 1, P)
  relidx = rel_idx.astype(jnp.int32).reshape(C, 1, P)
  pos, ln, rn = _make_tc_scores(C)(g3, g3, idx3, idx3, relidx, rel_diag)
  return pos.reshape(C, P), ln, rn


# speedup vs baseline: 1.2677x; 1.2677x over previous
import functools

import numpy as np

import jax
import jax.numpy as jnp
from jax import lax
from jax.experimental import pallas as pl
from jax.experimental.pallas import tpu as pltpu
from jax.experimental.pallas import tpu_sc as plsc

D = 64
R = 16
P = 512

_TW = 8192
_QW = _TW // 4
_MASK_HI = np.uint32(0xFFFF0000)
_RND = np.uint32(0x8000)

_NC = 2
_NS = 16
_NW = _NC * _NS
_GCHUNK = 128
_PHASE = 512


def _pack_pair(a_u32, b_u32):
  return (((a_u32 + _RND) & _MASK_HI)
          | ((b_u32 + _RND) >> np.uint32(16)))


def _transpose_body(x_ref, o_ref):
  u = lax.bitcast_convert_type(x_ref[...], jnp.uint32)
  lo = _pack_pair(u[:, :_QW], u[:, _QW:2 * _QW])
  hi = _pack_pair(u[:, 2 * _QW:3 * _QW], u[:, 3 * _QW:])
  z = jnp.concatenate([lo, hi], axis=0)
  o_ref[...] = lax.bitcast_convert_type(z, jnp.float32).T


def _make_pack(n_cols):
  n_blocks = (n_cols + _TW - 1) // _TW
  return pl.pallas_call(
      _transpose_body,
      grid=(n_blocks,),
      in_specs=[pl.BlockSpec((D, _TW), lambda j: (0, j))],
      out_specs=pl.BlockSpec((_QW, 2 * D), lambda j: (j, 0)),
      out_shape=jax.ShapeDtypeStruct((n_blocks * _QW, 2 * D), jnp.float32),
      compiler_params=pltpu.CompilerParams(
          dimension_semantics=("parallel",)),
  )


def _make_sc_gather(n_idx):
  b_per_w = n_idx // _NW
  mesh = plsc.VectorSubcoreMesh(core_axis_name="c", subcore_axis_name="s")

  @functools.partial(
      pl.kernel,
      mesh=mesh,
      out_type=jax.ShapeDtypeStruct((n_idx, 2 * D), jnp.float32),
      scratch_types=[
          pltpu.VMEM((b_per_w,), jnp.int32),
          pltpu.VMEM((_PHASE, 2 * D), jnp.float32),
          pltpu.SemaphoreType.DMA,
      ],
  )
  def gather_kernel(table_hbm, idx_hbm, out_hbm, idx_v, rows_v, sem):
    wid = lax.axis_index("s") * _NC + lax.axis_index("c")
    base = wid * b_per_w
    pltpu.sync_copy(idx_hbm.at[pl.ds(base, b_per_w)], idx_v)
    for ph in range(b_per_w // _PHASE):
      copies = []
      for j in range(_PHASE // _GCHUNK):
        isl = pl.ds(ph * _PHASE + j * _GCHUNK, _GCHUNK)
        rsl = pl.ds(j * _GCHUNK, _GCHUNK)
        copies.append(
            pltpu.async_copy(table_hbm.at[idx_v.at[isl]], rows_v.at[rsl],
                             sem))
      for c in copies:
        c.wait()
      pltpu.sync_copy(rows_v, out_hbm.at[pl.ds(base + ph * _PHASE, _PHASE)])

  return gather_kernel


def _unpack_side(g, idx):
  half = ((idx >> 12) & 1)[:, None] == 1
  slot = ((idx >> 11) & 1)[:, None] == 1
  gh = jnp.where(half, g[:, D:], g[:, :D])
  u = lax.bitcast_convert_type(gh, jnp.uint32)
  va = lax.bitcast_convert_type(u & _MASK_HI, jnp.float32)
  vb = lax.bitcast_convert_type(u << np.uint32(16), jnp.float32)
  return jnp.where(slot, vb, va)


def _score_body(lhs_ref, rhs_ref, lidx_ref, ridx_ref, relidx_ref, diag_ref,
                pos_ref, ln_ref, rn_ref):
  lhs = _unpack_side(lhs_ref[0], lidx_ref[0, 0])
  rhs = _unpack_side(rhs_ref[0], ridx_ref[0, 0])
  rel = relidx_ref[0, 0]
  onehot = (rel[:, None] == lax.broadcasted_iota(jnp.int32, (P, R), 1))
  diag = jnp.dot(onehot.astype(jnp.float32), diag_ref[...],
                 preferred_element_type=jnp.float32)
  rhs_t = rhs * diag
  pos_ref[0] = jnp.sum(lhs * rhs_t, axis=1)[None, :]
  lhs_bf = lhs.astype(jnp.bfloat16)
  rhs_bf = rhs_t.astype(jnp.bfloat16)
  dn = (((1,), (1,)), ((), ()))
  ln_ref[0] = lax.dot_general(rhs_bf, lhs_bf, dn,
                              preferred_element_type=jnp.float32)
  rn_ref[0] = lax.dot_general(lhs_bf, rhs_bf, dn,
                              preferred_element_type=jnp.float32)


def _make_tc_scores(C):
  return pl.pallas_call(
      _score_body,
      grid=(C,),
      in_specs=[
          pl.BlockSpec((1, P, 2 * D), lambda c: (c, 0, 0)),
          pl.BlockSpec((1, P, 2 * D), lambda c, _C=C: (_C + c, 0, 0)),
          pl.BlockSpec((1, 1, P), lambda c: (c, 0, 0)),
          pl.BlockSpec((1, 1, P), lambda c, _C=C: (_C + c, 0, 0)),
          pl.BlockSpec((1, 1, P), lambda c: (c, 0, 0)),
          pl.BlockSpec((R, D), lambda c: (0, 0)),
      ],
      out_specs=[
          pl.BlockSpec((1, 1, P), lambda c: (c, 0, 0)),
          pl.BlockSpec((1, P, P), lambda c: (c, 0, 0)),
          pl.BlockSpec((1, P, P), lambda c: (c, 0, 0)),
      ],
      out_shape=[
          jax.ShapeDtypeStruct((C, 1, P), jnp.float32),
          jax.ShapeDtypeStruct((C, P, P), jnp.float32),
          jax.ShapeDtypeStruct((C, P, P), jnp.float32),
      ],
      compiler_params=pltpu.CompilerParams(
          dimension_semantics=("parallel",)),
  )


@jax.jit
def kernel(emb, rel_diag, lhs_idx, rhs_idx, rel_idx):
  B = lhs_idx.shape[0]
  C = B // P
  V = emb.shape[0]
  packed = _make_pack(V)(emb.T)
  idx_all = jnp.concatenate([lhs_idx, rhs_idx]).astype(jnp.int32)
  gidx = ((idx_all >> 13) << 11) | (idx_all & 2047)
  gathered = _make_sc_gather(2 * B)(packed, gidx)
  g3 = gathered.reshape(2 * C, P, 2 * D)
  idx3 = idx_all.reshape(2 * C, 1, P)
  relidx = rel_idx.astype(jnp.int32).reshape(C, 1, P)
  pos, ln, rn = _make_tc_scores(C)(g3, g3, idx3, idx3, relidx, rel_diag)
  return pos.reshape(C, P), ln, rn


# speedup vs baseline: 1.4568x; 1.1492x over previous
import functools

import numpy as np

import jax
import jax.numpy as jnp
from jax import lax
from jax.experimental import pallas as pl
from jax.experimental.pallas import tpu as pltpu
from jax.experimental.pallas import tpu_sc as plsc

D = 64
R = 16
P = 512

_TW = 16384
_LOG_TW = 14
_QW = _TW // 4
_MASK_HI = np.uint32(0xFFFF0000)
_RND = np.uint32(0x8000)

_NC = 2
_NS = 16
_NW = _NC * _NS
_GCHUNK = 128
_PHASE = 512


def _pack_pair(a_u32, b_u32):
  return (((a_u32 + _RND) & _MASK_HI)
          | ((b_u32 + _RND) >> np.uint32(16)))


def _transpose_body(x_ref, o_ref):
  u = lax.bitcast_convert_type(x_ref[...], jnp.uint32)
  lo = _pack_pair(u[:, :_QW], u[:, _QW:2 * _QW])
  hi = _pack_pair(u[:, 2 * _QW:3 * _QW], u[:, 3 * _QW:])
  z = jnp.concatenate([lo, hi], axis=0)
  o_ref[...] = lax.bitcast_convert_type(z, jnp.float32).T


def _make_pack(n_cols):
  n_blocks = (n_cols + _TW - 1) // _TW
  return pl.pallas_call(
      _transpose_body,
      grid=(n_blocks,),
      in_specs=[pl.BlockSpec((D, _TW), lambda j: (0, j))],
      out_specs=pl.BlockSpec((_QW, 2 * D), lambda j: (j, 0)),
      out_shape=jax.ShapeDtypeStruct((n_blocks * _QW, 2 * D), jnp.float32),
      compiler_params=pltpu.CompilerParams(
          dimension_semantics=("parallel",)),
  )


def _make_sc_gather(n_idx):
  b_per_w = n_idx // _NW
  mesh = plsc.VectorSubcoreMesh(core_axis_name="c", subcore_axis_name="s")

  @functools.partial(
      pl.kernel,
      mesh=mesh,
      out_type=jax.ShapeDtypeStruct((n_idx, 2 * D), jnp.float32),
      scratch_types=[
          pltpu.VMEM((b_per_w,), jnp.int32),
          pltpu.VMEM((_PHASE, 2 * D), jnp.float32),
          pltpu.SemaphoreType.DMA,
      ],
  )
  def gather_kernel(table_hbm, idx_hbm, out_hbm, idx_v, rows_v, sem):
    wid = lax.axis_index("s") * _NC + lax.axis_index("c")
    base = wid * b_per_w
    pltpu.sync_copy(idx_hbm.at[pl.ds(base, b_per_w)], idx_v)
    for ph in range(b_per_w // _PHASE):
      copies = []
      for j in range(_PHASE // _GCHUNK):
        isl = pl.ds(ph * _PHASE + j * _GCHUNK, _GCHUNK)
        rsl = pl.ds(j * _GCHUNK, _GCHUNK)
        copies.append(
            pltpu.async_copy(table_hbm.at[idx_v.at[isl]], rows_v.at[rsl],
                             sem))
      for c in copies:
        c.wait()
      pltpu.sync_copy(rows_v, out_hbm.at[pl.ds(base + ph * _PHASE, _PHASE)])

  return gather_kernel


def _unpack_side(g, idx):
  half = ((idx >> (_LOG_TW - 1)) & 1)[:, None] == 1
  slot = ((idx >> (_LOG_TW - 2)) & 1)[:, None] == 1
  gh = jnp.where(half, g[:, D:], g[:, :D])
  u = lax.bitcast_convert_type(gh, jnp.uint32)
  va = lax.bitcast_convert_type(u & _MASK_HI, jnp.float32)
  vb = lax.bitcast_convert_type(u << np.uint32(16), jnp.float32)
  return jnp.where(slot, vb, va)


def _score_body(lhs_ref, rhs_ref, lidx_ref, ridx_ref, relidx_ref, diag_ref,
                pos_ref, ln_ref, rn_ref):
  lhs = _unpack_side(lhs_ref[0], lidx_ref[0, 0])
  rhs = _unpack_side(rhs_ref[0], ridx_ref[0, 0])
  rel = relidx_ref[0, 0]
  onehot = (rel[:, None] == lax.broadcasted_iota(jnp.int32, (P, R), 1))
  diag = jnp.dot(onehot.astype(jnp.float32), diag_ref[...],
                 preferred_element_type=jnp.float32)
  rhs_t = rhs * diag
  pos_ref[0] = jnp.sum(lhs * rhs_t, axis=1)[None, :]
  lhs_bf = lhs.astype(jnp.bfloat16)
  rhs_bf = rhs_t.astype(jnp.bfloat16)
  dn = (((1,), (1,)), ((), ()))
  ln_ref[0] = lax.dot_general(rhs_bf, lhs_bf, dn,
                              preferred_element_type=jnp.float32)
  rn_ref[0] = lax.dot_general(lhs_bf, rhs_bf, dn,
                              preferred_element_type=jnp.float32)


def _make_tc_scores(C):
  return pl.pallas_call(
      _score_body,
      grid=(C,),
      in_specs=[
          pl.BlockSpec((1, P, 2 * D), lambda c: (c, 0, 0)),
          pl.BlockSpec((1, P, 2 * D), lambda c, _C=C: (_C + c, 0, 0)),
          pl.BlockSpec((1, 1, P), lambda c: (c, 0, 0)),
          pl.BlockSpec((1, 1, P), lambda c, _C=C: (_C + c, 0, 0)),
          pl.BlockSpec((1, 1, P), lambda c: (c, 0, 0)),
          pl.BlockSpec((R, D), lambda c: (0, 0)),
      ],
      out_specs=[
          pl.BlockSpec((1, 1, P), lambda c: (c, 0, 0)),
          pl.BlockSpec((1, P, P), lambda c: (c, 0, 0)),
          pl.BlockSpec((1, P, P), lambda c: (c, 0, 0)),
      ],
      out_shape=[
          jax.ShapeDtypeStruct((C, 1, P), jnp.float32),
          jax.ShapeDtypeStruct((C, P, P), jnp.float32),
          jax.ShapeDtypeStruct((C, P, P), jnp.float32),
      ],
      compiler_params=pltpu.CompilerParams(
          dimension_semantics=("parallel",)),
  )


@jax.jit
def kernel(emb, rel_diag, lhs_idx, rhs_idx, rel_idx):
  B = lhs_idx.shape[0]
  C = B // P
  V = emb.shape[0]
  packed = _make_pack(V)(emb.T)
  idx_all = jnp.concatenate([lhs_idx, rhs_idx]).astype(jnp.int32)
  gidx = (((idx_all >> _LOG_TW) << (_LOG_TW - 2))
          | (idx_all & (_QW - 1)))
  gathered = _make_sc_gather(2 * B)(packed, gidx)
  g3 = gathered.reshape(2 * C, P, 2 * D)
  idx3 = idx_all.reshape(2 * C, 1, P)
  relidx = rel_idx.astype(jnp.int32).reshape(C, 1, P)
  pos, ln, rn = _make_tc_scores(C)(g3, g3, idx3, idx3, relidx, rel_diag)
  return pos.reshape(C, P), ln, rn


# speedup vs baseline: 1.4945x; 1.0259x over previous
import functools

import numpy as np

import jax
import jax.numpy as jnp
from jax import lax
from jax.experimental import pallas as pl
from jax.experimental.pallas import tpu as pltpu
from jax.experimental.pallas import tpu_sc as plsc

D = 64
R = 16
P = 512

_TW = 32768
_LOG_TW = 15
_QW = _TW // 4
_MASK_HI = np.uint32(0xFFFF0000)
_RND = np.uint32(0x8000)

_NC = 2
_NS = 16
_NW = _NC * _NS
_GCHUNK = 128
_PHASE = 512


def _pack_pair(a_u32, b_u32):
  return (((a_u32 + _RND) & _MASK_HI)
          | ((b_u32 + _RND) >> np.uint32(16)))


def _transpose_body(x_ref, o_ref):
  u = lax.bitcast_convert_type(x_ref[...], jnp.uint32)
  lo = _pack_pair(u[:, :_QW], u[:, _QW:2 * _QW])
  hi = _pack_pair(u[:, 2 * _QW:3 * _QW], u[:, 3 * _QW:])
  z = jnp.concatenate([lo, hi], axis=0)
  o_ref[...] = lax.bitcast_convert_type(z, jnp.float32).T


def _make_pack(n_cols):
  n_blocks = (n_cols + _TW - 1) // _TW
  return pl.pallas_call(
      _transpose_body,
      grid=(n_blocks,),
      in_specs=[pl.BlockSpec((D, _TW), lambda j: (0, j))],
      out_specs=pl.BlockSpec((_QW, 2 * D), lambda j: (j, 0)),
      out_shape=jax.ShapeDtypeStruct((n_blocks * _QW, 2 * D), jnp.float32),
      compiler_params=pltpu.CompilerParams(
          dimension_semantics=("parallel",)),
  )


def _make_sc_gather(n_idx):
  b_per_w = n_idx // _NW
  mesh = plsc.VectorSubcoreMesh(core_axis_name="c", subcore_axis_name="s")

  @functools.partial(
      pl.kernel,
      mesh=mesh,
      out_type=jax.ShapeDtypeStruct((n_idx, 2 * D), jnp.float32),
      scratch_types=[
          pltpu.VMEM((b_per_w,), jnp.int32),
          pltpu.VMEM((_PHASE, 2 * D), jnp.float32),
          pltpu.SemaphoreType.DMA,
      ],
  )
  def gather_kernel(table_hbm, idx_hbm, out_hbm, idx_v, rows_v, sem):
    wid = lax.axis_index("s") * _NC + lax.axis_index("c")
    base = wid * b_per_w
    pltpu.sync_copy(idx_hbm.at[pl.ds(base, b_per_w)], idx_v)
    for ph in range(b_per_w // _PHASE):
      copies = []
      for j in range(_PHASE // _GCHUNK):
        isl = pl.ds(ph * _PHASE + j * _GCHUNK, _GCHUNK)
        rsl = pl.ds(j * _GCHUNK, _GCHUNK)
        copies.append(
            pltpu.async_copy(table_hbm.at[idx_v.at[isl]], rows_v.at[rsl],
                             sem))
      for c in copies:
        c.wait()
      pltpu.sync_copy(rows_v, out_hbm.at[pl.ds(base + ph * _PHASE, _PHASE)])

  return gather_kernel


def _unpack_side(g, idx):
  half = ((idx >> (_LOG_TW - 1)) & 1)[:, None] == 1
  slot = ((idx >> (_LOG_TW - 2)) & 1)[:, None] == 1
  gh = jnp.where(half, g[:, D:], g[:, :D])
  u = lax.bitcast_convert_type(gh, jnp.uint32)
  va = lax.bitcast_convert_type(u & _MASK_HI, jnp.float32)
  vb = lax.bitcast_convert_type(u << np.uint32(16), jnp.float32)
  return jnp.where(slot, vb, va)


def _score_body(lhs_ref, rhs_ref, lidx_ref, ridx_ref, relidx_ref, diag_ref,
                pos_ref, ln_ref, rn_ref):
  lhs = _unpack_side(lhs_ref[0], lidx_ref[0, 0])
  rhs = _unpack_side(rhs_ref[0], ridx_ref[0, 0])
  rel = relidx_ref[0, 0]
  onehot = (rel[:, None] == lax.broadcasted_iota(jnp.int32, (P, R), 1))
  diag = jnp.dot(onehot.astype(jnp.float32), diag_ref[...],
                 preferred_element_type=jnp.float32)
  rhs_t = rhs * diag
  pos_ref[0] = jnp.sum(lhs * rhs_t, axis=1)[None, :]
  lhs_bf = lhs.astype(jnp.bfloat16)
  rhs_bf = rhs_t.astype(jnp.bfloat16)
  dn = (((1,), (1,)), ((), ()))
  ln_ref[0] = lax.dot_general(rhs_bf, lhs_bf, dn,
                              preferred_element_type=jnp.float32)
  rn_ref[0] = lax.dot_general(lhs_bf, rhs_bf, dn,
                              preferred_element_type=jnp.float32)


def _make_tc_scores(C):
  return pl.pallas_call(
      _score_body,
      grid=(C,),
      in_specs=[
          pl.BlockSpec((1, P, 2 * D), lambda c: (c, 0, 0)),
          pl.BlockSpec((1, P, 2 * D), lambda c, _C=C: (_C + c, 0, 0)),
          pl.BlockSpec((1, 1, P), lambda c: (c, 0, 0)),
          pl.BlockSpec((1, 1, P), lambda c, _C=C: (_C + c, 0, 0)),
          pl.BlockSpec((1, 1, P), lambda c: (c, 0, 0)),
          pl.BlockSpec((R, D), lambda c: (0, 0)),
      ],
      out_specs=[
          pl.BlockSpec((1, 1, P), lambda c: (c, 0, 0)),
          pl.BlockSpec((1, P, P), lambda c: (c, 0, 0)),
          pl.BlockSpec((1, P, P), lambda c: (c, 0, 0)),
      ],
      out_shape=[
          jax.ShapeDtypeStruct((C, 1, P), jnp.float32),
          jax.ShapeDtypeStruct((C, P, P), jnp.float32),
          jax.ShapeDtypeStruct((C, P, P), jnp.float32),
      ],
      compiler_params=pltpu.CompilerParams(
          dimension_semantics=("parallel",)),
  )


@jax.jit
def kernel(emb, rel_diag, lhs_idx, rhs_idx, rel_idx):
  B = lhs_idx.shape[0]
  C = B // P
  V = emb.shape[0]
  packed = _make_pack(V)(emb.T)
  idx_all = jnp.concatenate([lhs_idx, rhs_idx]).astype(jnp.int32)
  gidx = (((idx_all >> _LOG_TW) << (_LOG_TW - 2))
          | (idx_all & (_QW - 1)))
  gathered = _make_sc_gather(2 * B)(packed, gidx)
  g3 = gathered.reshape(2 * C, P, 2 * D)
  idx3 = idx_all.reshape(2 * C, 1, P)
  relidx = rel_idx.astype(jnp.int32).reshape(C, 1, P)
  pos, ln, rn = _make_tc_scores(C)(g3, g3, idx3, idx3, relidx, rel_diag)
  return pos.reshape(C, P), ln, rn
